# baseline (speedup 1.0000x reference)
import functools

import jax
import jax.numpy as jnp
from jax import lax
from jax.experimental import pallas as pl
from jax.experimental.pallas import tpu as pltpu
from jax.experimental.pallas import tpu_sc as plsc

C = 128
C2 = 256
NNODES = 50000
KNB = 9
NW = 32
CPN = 14
NCH = 112
NCHP = NCH + 4
NPW = CPN * NCH
NPAD = NW * NPW
ROWS = 128
GRP = 4
BN = 1024
NBLK = NPAD // BN
LANES = 16


def _tc_transpose(x2dp):
    def k1(x_ref, o_ref):
        o_ref[...] = x_ref[...].T

    return pl.pallas_call(
        k1,
        grid=(NBLK,),
        in_specs=[pl.BlockSpec((C, BN), lambda i: (0, i))],
        out_specs=pl.BlockSpec((BN, C), lambda i: (i, 0)),
        out_shape=jax.ShapeDtypeStruct((NPAD, C), jnp.float32),
    )(x2dp)


def _sc_gather_maxdiff(xT, idxJ, idxI):
    mesh = plsc.VectorSubcoreMesh(core_axis_name="c", subcore_axis_name="s")

    @functools.partial(
        pl.kernel,
        mesh=mesh,
        out_type=jax.ShapeDtypeStruct((NPAD, C), jnp.float32),
        scratch_types=[
            pltpu.VMEM((NCHP, ROWS), jnp.int32),
            pltpu.VMEM((NCHP, ROWS), jnp.int32),
            pltpu.VMEM((ROWS, C), jnp.float32),
            pltpu.VMEM((ROWS, C), jnp.float32),
            pltpu.VMEM((ROWS, C), jnp.float32),
            pltpu.VMEM((ROWS, C), jnp.float32),
            pltpu.VMEM((GRP * CPN, C), jnp.float32),
            pltpu.SemaphoreType.DMA,
            pltpu.SemaphoreType.DMA,
        ],
    )
    def k2(xT_hbm, idxJ_hbm, idxI_hbm, out_hbm,
           idxj_v, idxi_v, rowsj0, rowsi0, rowsj1, rowsi1, xj_v, sem0, sem1):
        wid = lax.axis_index("s") * 2 + lax.axis_index("c")
        bufs = ((rowsj0, rowsi0, sem0), (rowsj1, rowsi1, sem1))
        pltpu.sync_copy(idxJ_hbm.at[wid], idxj_v)
        pltpu.sync_copy(idxI_hbm.at[wid], idxi_v)

        def issue(j, buf):
            rj, ri, sem = bufs[buf]
            pltpu.async_copy(xT_hbm.at[idxj_v.at[j]], rj, sem)
            pltpu.async_copy(xT_hbm.at[idxi_v.at[j]], ri, sem)

        def drain(buf):
            rj, ri, sem = bufs[buf]
            dummy = xT_hbm.at[pl.ds(0, ROWS)]
            pltpu.make_async_copy(dummy, rj, sem).wait()
            pltpu.make_async_copy(dummy, ri, sem).wait()

        issue(0, 0)
        issue(1, 1)

        def group(jj, carry):
            for bb in range(GRP):
                buf = bb % 2
                rj, ri, _ = bufs[buf]
                j = jj * GRP + bb
                drain(buf)

                def node(ln, c2):
                    base = ln * KNB
                    for v in range(C // LANES):
                        sl = pl.ds(v * LANES, LANES)
                        acc = rj[base, sl] - ri[base, sl]
                        for kk in range(1, KNB):
                            acc = jnp.maximum(
                                acc, rj[base + kk, sl] - ri[base + kk, sl])
                        xj_v[bb * CPN + ln, sl] = acc
                    return c2

                lax.fori_loop(0, CPN, node, 0)
                issue(j + 2, buf)
            pltpu.sync_copy(
                xj_v, out_hbm.at[pl.ds(wid * NPW + jj * (GRP * CPN), GRP * CPN)])
            return carry

        lax.fori_loop(0, NCH // GRP, group, 0)
        drain(0)
        drain(1)

    return k2(xT, idxJ, idxI)


def _tc_stats(x2dp, xjT, W, bcol, gcol, betacol):
    def k3(x_ref, xjT_ref, W_ref, b_ref, g_ref, beta_ref,
           W2_ref, b2_ref, gxx, gxj, gjj, sx, sj):
        pid = pl.program_id(0)

        @pl.when(pid == 0)
        def _init():
            gxx[...] = jnp.zeros_like(gxx)
            gxj[...] = jnp.zeros_like(gxj)
            gjj[...] = jnp.zeros_like(gjj)
            sx[...] = jnp.zeros_like(sx)
            sj[...] = jnp.zeros_like(sj)

        xb = x_ref[...]
        jb = xjT_ref[...]
        f32 = jnp.float32
        gxx[...] += lax.dot_general(xb, xb, (((1,), (1,)), ((), ())),
                                    preferred_element_type=f32, precision=lax.Precision.HIGHEST)
        gxj[...] += lax.dot_general(xb, jb, (((1,), (0,)), ((), ())),
                                    preferred_element_type=f32, precision=lax.Precision.HIGHEST)
        gjj[...] += lax.dot_general(jb, jb, (((0,), (0,)), ((), ())),
                                    preferred_element_type=f32, precision=lax.Precision.HIGHEST)
        sx[...] += jnp.sum(xb, axis=1, keepdims=True)
        sj[...] += jnp.sum(jb, axis=0, keepdims=True)

        @pl.when(pid == NBLK - 1)
        def _fold():
            top = jnp.concatenate([gxx[...], gxj[...]], axis=1)
            bot = jnp.concatenate([gxj[...].T, gjj[...]], axis=1)
            G = jnp.concatenate([top, bot], axis=0)
            m = jnp.concatenate([sx[...], sj[...].T], axis=0) * (1.0 / NNODES)
            cov = G * (1.0 / NNODES) - lax.dot_general(
                m, m, (((1,), (1,)), ((), ())), preferred_element_type=f32, precision=lax.Precision.HIGHEST)
            Wf = W_ref[...]
            t = jnp.dot(Wf, cov, preferred_element_type=f32, precision=lax.Precision.HIGHEST)
            var = jnp.sum(t * Wf, axis=1, keepdims=True)
            mean = jnp.dot(Wf, m, preferred_element_type=f32, precision=lax.Precision.HIGHEST) + b_ref[...]
            a = g_ref[...] * lax.rsqrt(var + 1e-5)
            W2_ref[...] = a * Wf
            b2_ref[...] = a * (b_ref[...] - mean) + beta_ref[...]

    const = lambda i: (0, 0)
    return pl.pallas_call(
        k3,
        grid=(NBLK,),
        in_specs=[
            pl.BlockSpec((C, BN), lambda i: (0, i)),
            pl.BlockSpec((BN, C), lambda i: (i, 0)),
            pl.BlockSpec((C2, C2), const),
            pl.BlockSpec((C2, 1), const),
            pl.BlockSpec((C2, 1), const),
            pl.BlockSpec((C2, 1), const),
        ],
        out_specs=[
            pl.BlockSpec((C2, C2), const),
            pl.BlockSpec((C2, 1), const),
        ],
        out_shape=[
            jax.ShapeDtypeStruct((C2, C2), jnp.float32),
            jax.ShapeDtypeStruct((C2, 1), jnp.float32),
        ],
        scratch_shapes=[
            pltpu.VMEM((C, C), jnp.float32),
            pltpu.VMEM((C, C), jnp.float32),
            pltpu.VMEM((C, C), jnp.float32),
            pltpu.VMEM((C, 1), jnp.float32),
            pltpu.VMEM((1, C), jnp.float32),
        ],
    )(x2dp, xjT, W, bcol, gcol, betacol)


def _tc_final(x2dp, xjT, W2, b2):
    def k4(x_ref, xjT_ref, W2_ref, b2_ref, o_ref):
        acc = jnp.dot(W2_ref[:, :C], x_ref[...],
                      preferred_element_type=jnp.float32, precision=lax.Precision.HIGHEST)
        acc = acc + lax.dot_general(W2_ref[:, C:], xjT_ref[...],
                                    (((1,), (1,)), ((), ())),
                                    preferred_element_type=jnp.float32, precision=lax.Precision.HIGHEST)
        o_ref[...] = jnp.maximum(acc + b2_ref[...], 0.0)

    return pl.pallas_call(
        k4,
        grid=(NBLK,),
        in_specs=[
            pl.BlockSpec((C, BN), lambda i: (0, i)),
            pl.BlockSpec((BN, C), lambda i: (i, 0)),
            pl.BlockSpec((C2, C2), lambda i: (0, 0)),
            pl.BlockSpec((C2, 1), lambda i: (0, 0)),
        ],
        out_specs=pl.BlockSpec((C2, BN), lambda i: (0, i)),
        out_shape=jax.ShapeDtypeStruct((C2, NNODES), jnp.float32),
    )(x2dp, xjT, W2, b2)


def kernel(x, edge_index, W, b, gamma, beta):
    x2d = x.reshape(C, NNODES)
    x2dp = jnp.pad(x2d, ((0, 0), (0, NPAD - NNODES)))
    e = edge_index.reshape(2, NNODES, KNB).astype(jnp.int32)
    ep = jnp.pad(e, ((0, 0), (0, NPAD - NNODES), (0, 0)))
    er = ep.reshape(2, NW, NCH, CPN * KNB)
    er = jnp.pad(er, ((0, 0), (0, 0), (0, NCHP - NCH), (0, ROWS - CPN * KNB)))

    Wp = jnp.concatenate([W[:, 0::2], W[:, 1::2]], axis=1)

    xT = _tc_transpose(x2dp)
    xjT = _sc_gather_maxdiff(xT, er[0], er[1])
    W2, b2 = _tc_stats(x2dp, xjT, Wp,
                       b.reshape(C2, 1), gamma.reshape(C2, 1),
                       beta.reshape(C2, 1))
    y = _tc_final(x2dp, xjT, W2, b2)
    return y.reshape(1, C2, NNODES, 1)


# speedup vs baseline: 1.5495x; 1.5495x over previous
import functools

import jax
import jax.numpy as jnp
from jax import lax
from jax.experimental import pallas as pl
from jax.experimental.pallas import tpu as pltpu
from jax.experimental.pallas import tpu_sc as plsc

C = 128
C2 = 256
NNODES = 50000
KNB = 9
NW = 32
CPN = 14
NCH = 112
NCHP = NCH + 4
NPW = CPN * NCH
NPAD = NW * NPW
ROWS = 128
GRP = 4
BN = 1024
NBLK = NPAD // BN
LANES = 16
CW = C // 2


def _tc_transpose(x2dp):
    def k1(x_ref, o_ref):
        o_ref[...] = x_ref[...].T

    return pl.pallas_call(
        k1,
        grid=(NBLK,),
        in_specs=[pl.BlockSpec((C, BN), lambda i: (0, i))],
        out_specs=pl.BlockSpec((BN, C), lambda i: (i, 0)),
        out_shape=jax.ShapeDtypeStruct((NPAD, C), jnp.float32),
    )(x2dp)


def _sc_gather_maxdiff(xT, idxJ, idxI):
    mesh = plsc.VectorSubcoreMesh(core_axis_name="c", subcore_axis_name="s")

    @functools.partial(
        pl.kernel,
        mesh=mesh,
        out_type=jax.ShapeDtypeStruct((NPAD, C), jnp.float32),
        scratch_types=[
            pltpu.VMEM((NCHP, ROWS), jnp.int32),
            pltpu.VMEM((NCHP, ROWS), jnp.int32),
            pltpu.VMEM((ROWS, C), jnp.float32),
            pltpu.VMEM((ROWS, C), jnp.float32),
            pltpu.VMEM((ROWS, C), jnp.float32),
            pltpu.VMEM((ROWS, C), jnp.float32),
            pltpu.VMEM((GRP * CPN, C), jnp.float32),
            pltpu.SemaphoreType.DMA,
            pltpu.SemaphoreType.DMA,
        ],
    )
    def k2(xT_hbm, idxJ_hbm, idxI_hbm, out_hbm,
           idxj_v, idxi_v, rowsj0, rowsi0, rowsj1, rowsi1, xj_v, sem0, sem1):
        wid = lax.axis_index("s") * 2 + lax.axis_index("c")
        bufs = ((rowsj0, rowsi0, sem0), (rowsj1, rowsi1, sem1))
        pltpu.sync_copy(idxJ_hbm.at[wid], idxj_v)
        pltpu.sync_copy(idxI_hbm.at[wid], idxi_v)

        def issue(j, buf):
            rj, ri, sem = bufs[buf]
            pltpu.async_copy(xT_hbm.at[idxj_v.at[j]], rj, sem)
            pltpu.async_copy(xT_hbm.at[idxi_v.at[j]], ri, sem)

        def drain(buf):
            rj, ri, sem = bufs[buf]
            dummy = xT_hbm.at[pl.ds(0, ROWS)]
            pltpu.make_async_copy(dummy, rj, sem).wait()
            pltpu.make_async_copy(dummy, ri, sem).wait()

        def group(jj, carry):
            for bb in range(GRP):
                buf = bb % 2
                rj, ri, _ = bufs[buf]
                j = jj * GRP + bb
                issue(j, buf)
                drain(buf)

                def node(ln, c2):
                    base = ln * KNB
                    for v in range(C // LANES):
                        sl = pl.ds(v * LANES, LANES)
                        acc = rj[base, sl] - ri[base, sl]
                        for kk in range(1, KNB):
                            acc = jnp.maximum(
                                acc, rj[base + kk, sl] - ri[base + kk, sl])
                        xj_v[bb * CPN + ln, sl] = acc
                    return c2

                lax.fori_loop(0, CPN, node, 0)
            pltpu.sync_copy(
                xj_v, out_hbm.at[pl.ds(wid * NPW + jj * (GRP * CPN), GRP * CPN)])
            return carry

        lax.fori_loop(0, NCH // GRP, group, 0)

    return k2(xT, idxJ, idxI)


def _tc_stats(x2dp, xjT, W, bcol, gcol, betacol):
    def k3(x_ref, xjT_ref, W_ref, b_ref, g_ref, beta_ref,
           W2_ref, b2_ref, gxx, gxj, gjj, sx, sj):
        pid = pl.program_id(0)

        @pl.when(pid == 0)
        def _init():
            gxx[...] = jnp.zeros_like(gxx)
            gxj[...] = jnp.zeros_like(gxj)
            gjj[...] = jnp.zeros_like(gjj)
            sx[...] = jnp.zeros_like(sx)
            sj[...] = jnp.zeros_like(sj)

        xb = x_ref[...]
        jb = xjT_ref[...]
        f32 = jnp.float32
        gxx[...] += lax.dot_general(xb, xb, (((1,), (1,)), ((), ())),
                                    preferred_element_type=f32)
        gxj[...] += lax.dot_general(xb, jb, (((1,), (0,)), ((), ())),
                                    preferred_element_type=f32)
        gjj[...] += lax.dot_general(jb, jb, (((0,), (0,)), ((), ())),
                                    preferred_element_type=f32)
        sx[...] += jnp.sum(xb, axis=1, keepdims=True)
        sj[...] += jnp.sum(jb, axis=0, keepdims=True)

        @pl.when(pid == NBLK - 1)
        def _fold():
            top = jnp.concatenate([gxx[...], gxj[...]], axis=1)
            bot = jnp.concatenate([gxj[...].T, gjj[...]], axis=1)
            G = jnp.concatenate([top, bot], axis=0)
            m = jnp.concatenate([sx[...], sj[...].T], axis=0) * (1.0 / NNODES)
            cov = G * (1.0 / NNODES) - lax.dot_general(
                m, m, (((1,), (1,)), ((), ())), preferred_element_type=f32, precision=lax.Precision.HIGHEST)
            Wf = W_ref[...]
            t = jnp.dot(Wf, cov, preferred_element_type=f32, precision=lax.Precision.HIGHEST)
            var = jnp.sum(t * Wf, axis=1, keepdims=True)
            mean = jnp.dot(Wf, m, preferred_element_type=f32, precision=lax.Precision.HIGHEST) + b_ref[...]
            a = g_ref[...] * lax.rsqrt(var + 1e-5)
            W2_ref[...] = a * Wf
            b2_ref[...] = a * (b_ref[...] - mean) + beta_ref[...]

    const = lambda i: (0, 0)
    return pl.pallas_call(
        k3,
        grid=(NBLK,),
        in_specs=[
            pl.BlockSpec((C, BN), lambda i: (0, i)),
            pl.BlockSpec((BN, C), lambda i: (i, 0)),
            pl.BlockSpec((C2, C2), const),
            pl.BlockSpec((C2, 1), const),
            pl.BlockSpec((C2, 1), const),
            pl.BlockSpec((C2, 1), const),
        ],
        out_specs=[
            pl.BlockSpec((C2, C2), const),
            pl.BlockSpec((C2, 1), const),
        ],
        out_shape=[
            jax.ShapeDtypeStruct((C2, C2), jnp.float32),
            jax.ShapeDtypeStruct((C2, 1), jnp.float32),
        ],
        scratch_shapes=[
            pltpu.VMEM((C, C), jnp.float32),
            pltpu.VMEM((C, C), jnp.float32),
            pltpu.VMEM((C, C), jnp.float32),
            pltpu.VMEM((C, 1), jnp.float32),
            pltpu.VMEM((1, C), jnp.float32),
        ],
    )(x2dp, xjT, W, bcol, gcol, betacol)


def _tc_final(x2dp, xjT, W2, b2):
    def k4(x_ref, xjT_ref, W2_ref, b2_ref, o_ref):
        acc = jnp.dot(W2_ref[:, :C], x_ref[...],
                      preferred_element_type=jnp.float32, precision=lax.Precision.HIGHEST)
        acc = acc + lax.dot_general(W2_ref[:, C:], xjT_ref[...],
                                    (((1,), (1,)), ((), ())),
                                    preferred_element_type=jnp.float32, precision=lax.Precision.HIGHEST)
        o_ref[...] = jnp.maximum(acc + b2_ref[...], 0.0)

    return pl.pallas_call(
        k4,
        grid=(NBLK,),
        in_specs=[
            pl.BlockSpec((C, BN), lambda i: (0, i)),
            pl.BlockSpec((BN, C), lambda i: (i, 0)),
            pl.BlockSpec((C2, C2), lambda i: (0, 0)),
            pl.BlockSpec((C2, 1), lambda i: (0, 0)),
        ],
        out_specs=pl.BlockSpec((C2, BN), lambda i: (0, i)),
        out_shape=jax.ShapeDtypeStruct((C2, NNODES), jnp.float32),
    )(x2dp, xjT, W2, b2)


def kernel(x, edge_index, W, b, gamma, beta):
    x2d = x.reshape(C, NNODES)
    x2dp = jnp.pad(x2d, ((0, 0), (0, NPAD - NNODES)))
    e = edge_index.reshape(2, NNODES, KNB).astype(jnp.int32)
    ep = jnp.pad(e, ((0, 0), (0, NPAD - NNODES), (0, 0)))
    er = ep.reshape(2, NW, NCH, CPN * KNB)
    er = jnp.pad(er, ((0, 0), (0, 0), (0, NCHP - NCH), (0, ROWS - CPN * KNB)))

    Wp = jnp.concatenate([W[:, 0::2], W[:, 1::2]], axis=1)

    xT = _tc_transpose(x2dp)
    xjT = _sc_gather_maxdiff(xT, er[0], er[1])
    W2, b2 = _tc_stats(x2dp, xjT, Wp,
                       b.reshape(C2, 1), gamma.reshape(C2, 1),
                       beta.reshape(C2, 1))
    y = _tc_final(x2dp, xjT, W2, b2)
    return y.reshape(1, C2, NNODES, 1)
